# baseline (speedup 1.0000x reference)
import functools

import jax
import jax.numpy as jnp
from jax import lax
from jax.experimental import pallas as pl
from jax.experimental.pallas import tpu as pltpu
from jax.experimental.pallas import tpu_sc as plsc

TMR = 256


def _routing_metadata(expert_weights, top_k_indices, T, K, E):
    P = T * K
    e_flat = top_k_indices.reshape(P).astype(jnp.int32)
    w_flat = expert_weights.reshape(P)
    onehot = (e_flat[:, None] == jnp.arange(E, dtype=jnp.int32)[None, :]).astype(jnp.int32)
    csum = jnp.cumsum(onehot, axis=0)
    counts = csum[-1]
    rank = jnp.take_along_axis(csum, e_flat[:, None], axis=1)[:, 0] - 1
    padded_counts = ((counts + TMR - 1) // TMR) * TMR
    cum_padded = jnp.cumsum(padded_counts)
    padded_offsets = cum_padded - padded_counts
    pos = padded_offsets[e_flat] + rank
    J = P + E * TMR
    NT = J // TMR
    row_token = jnp.zeros((J,), jnp.int32).at[pos].set(
        jnp.arange(P, dtype=jnp.int32) // K)
    row_w = jnp.zeros((J,), jnp.float32).at[pos].set(w_flat)
    tile_starts = jnp.arange(NT, dtype=jnp.int32) * TMR
    tile_expert = jnp.minimum(
        jnp.searchsorted(cum_padded, tile_starts, side="right").astype(jnp.int32),
        E - 1)
    return row_token, row_w, tile_expert, pos, J, NT


def _on_slot(slot, nbuf, fn):
    for b in range(nbuf):
        pl.when(slot == b)(functools.partial(fn, b))


def _sc_row_gather(src, indices, n_out, width):
    info = plsc.get_sparse_core_info()
    NC, NS = info.num_cores, info.num_subcores
    NW = NC * NS
    rows_per_w = n_out // NW
    CH = 32
    nchunk = rows_per_w // CH
    NBUF, H = 3, 2
    mesh = plsc.VectorSubcoreMesh(core_axis_name="c", subcore_axis_name="s")

    @functools.partial(
        pl.kernel, mesh=mesh,
        out_type=jax.ShapeDtypeStruct((n_out, width // 128, 128), jnp.int32),
        scratch_types=[
            pltpu.VMEM((rows_per_w,), jnp.int32),
            pltpu.VMEM((NBUF, CH, width // 128, 128), jnp.int32),
        ] + [pltpu.SemaphoreType.DMA] * (2 * NBUF),
    )
    def gather_k(src_hbm, idx_hbm, out_hbm, idx_v, bufs, *sems):
        gsem = sems[:NBUF]
        ssem = sems[NBUF:]
        wid = lax.axis_index("s") * NC + lax.axis_index("c")
        base0 = wid * rows_per_w
        pltpu.sync_copy(idx_hbm.at[pl.ds(base0, rows_per_w)], idx_v)

        def body(v, _):
            @pl.when(v < nchunk)
            def _():
                def start(b):
                    @pl.when(v >= NBUF)
                    def _():
                        pltpu.make_async_copy(
                            bufs.at[b], out_hbm.at[pl.ds(base0, CH)],
                            ssem[b]).wait()
                    pltpu.async_copy(
                        src_hbm.at[idx_v.at[pl.ds(v * CH, CH)]], bufs.at[b],
                        gsem[b])
                _on_slot(lax.rem(v, NBUF), NBUF, start)

            @pl.when(v >= H)
            def _():
                cc = v - H

                def finish(b):
                    pltpu.make_async_copy(
                        src_hbm.at[idx_v.at[pl.ds(cc * CH, CH)]], bufs.at[b],
                        gsem[b]).wait()
                    pltpu.async_copy(
                        bufs.at[b], out_hbm.at[pl.ds(base0 + cc * CH, CH)],
                        ssem[b])
                _on_slot(lax.rem(cc, NBUF), NBUF, finish)
            return 0

        lax.fori_loop(0, nchunk + H, body, 0)
        for c in range(nchunk - NBUF, nchunk):
            pltpu.make_async_copy(
                bufs.at[c % NBUF], out_hbm.at[pl.ds(base0, CH)],
                ssem[c % NBUF]).wait()

    return gather_k(src, indices)


def _tc_matmul_body(te_ref, xg_ref, rw_ref, w_ref, b_ref, o_ref):
    mm = lax.dot_general(xg_ref[...], w_ref[0], (((1,), (1,)), ((), ())),
                         preferred_element_type=jnp.float32)
    o_ref[...] = (rw_ref[...] * (mm + b_ref[0])).astype(jnp.bfloat16)


def _tc_ragged_matmul(xg, row_w, tile_expert, Wb, b, J, NT, D, O, E):
    grid_spec = pltpu.PrefetchScalarGridSpec(
        num_scalar_prefetch=1,
        grid=(NT,),
        in_specs=[
            pl.BlockSpec((TMR, D), lambda i, te: (i, 0)),
            pl.BlockSpec((TMR, 1), lambda i, te: (i, 0)),
            pl.BlockSpec((1, O, D), lambda i, te: (te[i], 0, 0)),
            pl.BlockSpec((1, 1, O), lambda i, te: (te[i], 0, 0)),
        ],
        out_specs=pl.BlockSpec((TMR, O), lambda i, te: (i, 0)),
    )
    return pl.pallas_call(
        _tc_matmul_body,
        grid_spec=grid_spec,
        out_shape=jax.ShapeDtypeStruct((J, O), jnp.bfloat16),
    )(tile_expert, xg, row_w.reshape(J, 1), Wb, b.reshape(E, 1, O))


def _tc_pair_add_body(in_ref, o_ref):
    o_ref[...] = (in_ref[:, 0, :].astype(jnp.float32)
                  + in_ref[:, 1, :].astype(jnp.float32))


def _tc_pair_add(ygp, T, O):
    TM2 = 512
    return pl.pallas_call(
        _tc_pair_add_body,
        grid=(T // TM2,),
        in_specs=[pl.BlockSpec((TM2, 2, O), lambda i: (i, 0, 0))],
        out_specs=pl.BlockSpec((TM2, O), lambda i: (i, 0)),
        out_shape=jax.ShapeDtypeStruct((T, O), jnp.float32),
    )(ygp.reshape(T, 2, O))


def kernel(x, expert_weights, top_k_indices, W, b):
    B, S, D = x.shape
    E, O, _ = W.shape
    K = expert_weights.shape[-1]
    T = B * S

    xb = x.reshape(T, D).astype(jnp.bfloat16)
    x_bits = lax.bitcast_convert_type(
        xb.reshape(T, D // 256, 128, 2), jnp.int32)
    Wb = W.astype(jnp.bfloat16)

    row_token, row_w, tile_expert, pos, J, NT = _routing_metadata(
        expert_weights, top_k_indices, T, K, E)

    xg_bits = _sc_row_gather(x_bits, row_token, J, D // 2)
    xg = lax.bitcast_convert_type(xg_bits, jnp.bfloat16).reshape(J, D)
    yg = _tc_ragged_matmul(xg, row_w, tile_expert, Wb, b, J, NT, D, O, E)
    yg_bits = lax.bitcast_convert_type(
        yg.reshape(J, O // 256, 128, 2), jnp.int32)
    ygp_bits = _sc_row_gather(yg_bits, pos, T * K, O // 2)
    ygp = lax.bitcast_convert_type(ygp_bits, jnp.bfloat16).reshape(T * K, O)
    out = _tc_pair_add(ygp, T, O)
    return out.reshape(B, S, O)


# speedup vs baseline: 14.3515x; 14.3515x over previous
import functools

import jax
import jax.numpy as jnp
from jax import lax
from jax.experimental import pallas as pl
from jax.experimental.pallas import tpu as pltpu
from jax.experimental.pallas import tpu_sc as plsc

TMR = 256


def _routing_metadata(expert_weights, top_k_indices, T, K, E):
    P = T * K
    e_flat = top_k_indices.reshape(P).astype(jnp.int32)
    w_flat = expert_weights.reshape(P)
    onehot = (e_flat[:, None] == jnp.arange(E, dtype=jnp.int32)[None, :]).astype(jnp.int32)
    csum = jnp.cumsum(onehot, axis=0)
    counts = csum[-1]
    rank = jnp.take_along_axis(csum, e_flat[:, None], axis=1)[:, 0] - 1
    padded_counts = ((counts + TMR - 1) // TMR) * TMR
    cum_padded = jnp.cumsum(padded_counts)
    padded_offsets = cum_padded - padded_counts
    pos = padded_offsets[e_flat] + rank
    J = P + E * TMR
    NT = J // TMR
    row_token = jnp.zeros((J,), jnp.int32).at[pos].set(
        jnp.arange(P, dtype=jnp.int32) // K)
    row_w = jnp.zeros((J,), jnp.float32).at[pos].set(w_flat)
    tile_starts = jnp.arange(NT, dtype=jnp.int32) * TMR
    tile_expert = jnp.minimum(
        jnp.searchsorted(cum_padded, tile_starts, side="right").astype(jnp.int32),
        E - 1)
    pos2 = pos.reshape(T, K)
    return row_token, row_w, tile_expert, pos2[:, 0], pos2[:, 1], J, NT


def _pack_halves(lo_f32, hi_f32):
    lo_u = lax.bitcast_convert_type(lo_f32, jnp.uint32)
    hi_u = lax.bitcast_convert_type(hi_f32, jnp.uint32)
    lo_r = (lo_u + 0x8000) >> 16
    hi_r = (hi_u + 0x8000) & jnp.uint32(0xFFFF0000)
    return lax.bitcast_convert_type(lo_r | hi_r, jnp.int32)


def _unpack_halves(words_i32):
    u = lax.bitcast_convert_type(words_i32, jnp.uint32)
    lo = lax.bitcast_convert_type(u << 16, jnp.float32)
    hi = lax.bitcast_convert_type(u & jnp.uint32(0xFFFF0000), jnp.float32)
    return lo, hi


def _tc_pack_body(x_ref, o_ref):
    half = o_ref.shape[1]
    o_ref[...] = _pack_halves(x_ref[:, :half], x_ref[:, half:])


def _tc_pack(x2, T, D):
    TMP = 512
    return pl.pallas_call(
        _tc_pack_body,
        grid=(T // TMP,),
        in_specs=[pl.BlockSpec((TMP, D), lambda i: (i, 0))],
        out_specs=pl.BlockSpec((TMP, D // 2), lambda i: (i, 0)),
        out_shape=jax.ShapeDtypeStruct((T, D // 2), jnp.int32),
    )(x2)


def _sc_row_gather(src, indices, n_out, width):
    info = plsc.get_sparse_core_info()
    NC, NS = info.num_cores, info.num_subcores
    NW = NC * NS
    rows_per_w = n_out // NW
    CH = 32
    nchunk = rows_per_w // CH
    NBUF, H = 3, 2
    mesh = plsc.VectorSubcoreMesh(core_axis_name="c", subcore_axis_name="s")

    @functools.partial(
        pl.kernel, mesh=mesh,
        out_type=jax.ShapeDtypeStruct((n_out, width), jnp.int32),
        scratch_types=[
            pltpu.VMEM((nchunk, 1, CH), jnp.int32),
            pltpu.VMEM((NBUF, CH, width), jnp.int32),
        ] + [pltpu.SemaphoreType.DMA] * (2 * NBUF),
    )
    def gather_k(src_hbm, idx_hbm, out_hbm, idx_v, bufs, *sems):
        gsem = sems[:NBUF]
        ssem = sems[NBUF:]
        wid = lax.axis_index("s") * NC + lax.axis_index("c")
        base0 = wid * rows_per_w
        pltpu.sync_copy(idx_hbm.at[pl.ds(wid * nchunk, nchunk)], idx_v)

        def body(v, _):
            @pl.when(v < nchunk)
            def _():
                def start(b):
                    @pl.when(v >= NBUF)
                    def _():
                        pltpu.make_async_copy(
                            bufs.at[b], out_hbm.at[pl.ds(base0, CH)],
                            ssem[b]).wait()
                    pltpu.async_copy(
                        src_hbm.at[idx_v.at[v, 0]], bufs.at[b], gsem[b])
                _on_slot(lax.rem(v, NBUF), NBUF, start)

            @pl.when(v >= H)
            def _():
                cc = v - H

                def finish(b):
                    pltpu.make_async_copy(
                        src_hbm.at[idx_v.at[cc, 0]], bufs.at[b],
                        gsem[b]).wait()
                    pltpu.async_copy(
                        bufs.at[b], out_hbm.at[pl.ds(base0 + cc * CH, CH)],
                        ssem[b])
                _on_slot(lax.rem(cc, NBUF), NBUF, finish)
            return 0

        lax.fori_loop(0, nchunk + H, body, 0)
        for c in range(nchunk - NBUF, nchunk):
            pltpu.make_async_copy(
                bufs.at[c % NBUF], out_hbm.at[pl.ds(base0, CH)],
                ssem[c % NBUF]).wait()

    return gather_k(src.reshape(-1, width),
                    indices.reshape(NW * nchunk, 1, CH))


def _on_slot(slot, nbuf, fn):
    for b in range(nbuf):
        pl.when(slot == b)(functools.partial(fn, b))


def _tc_matmul_body(te_ref, xg_ref, rw_ref, w_ref, b_ref, o_ref):
    halfk = xg_ref.shape[1]
    lo, hi = _unpack_halves(xg_ref[...])
    w = w_ref[0]
    mm = lax.dot_general(lo.astype(jnp.bfloat16), w[:, :halfk],
                         (((1,), (1,)), ((), ())),
                         preferred_element_type=jnp.float32)
    mm = mm + lax.dot_general(hi.astype(jnp.bfloat16), w[:, halfk:],
                              (((1,), (1,)), ((), ())),
                              preferred_element_type=jnp.float32)
    y = rw_ref[...] * (mm + b_ref[0])
    halfo = o_ref.shape[1]
    o_ref[...] = _pack_halves(y[:, :halfo], y[:, halfo:])


def _tc_ragged_matmul(xg_bits, row_w, tile_expert, Wb, b, J, NT, D, O, E):
    grid_spec = pltpu.PrefetchScalarGridSpec(
        num_scalar_prefetch=1,
        grid=(NT,),
        in_specs=[
            pl.BlockSpec((TMR, D // 2), lambda i, te: (i, 0)),
            pl.BlockSpec((TMR, 1), lambda i, te: (i, 0)),
            pl.BlockSpec((1, O, D), lambda i, te: (te[i], 0, 0)),
            pl.BlockSpec((1, 1, O), lambda i, te: (te[i], 0, 0)),
        ],
        out_specs=pl.BlockSpec((TMR, O // 2), lambda i, te: (i, 0)),
    )
    return pl.pallas_call(
        _tc_matmul_body,
        grid_spec=grid_spec,
        out_shape=jax.ShapeDtypeStruct((J, O // 2), jnp.int32),
    )(tile_expert, xg_bits, row_w.reshape(J, 1), Wb, b.reshape(E, 1, O))


def _tc_combine_body(a_ref, b_ref, o_ref):
    alo, ahi = _unpack_halves(a_ref[...])
    blo, bhi = _unpack_halves(b_ref[...])
    half = a_ref.shape[1]
    o_ref[:, :half] = alo + blo
    o_ref[:, half:] = ahi + bhi


def _tc_combine(ygp0, ygp1, T, O):
    TMC = 512
    return pl.pallas_call(
        _tc_combine_body,
        grid=(T // TMC,),
        in_specs=[
            pl.BlockSpec((TMC, O // 2), lambda i: (i, 0)),
            pl.BlockSpec((TMC, O // 2), lambda i: (i, 0)),
        ],
        out_specs=pl.BlockSpec((TMC, O), lambda i: (i, 0)),
        out_shape=jax.ShapeDtypeStruct((T, O), jnp.float32),
    )(ygp0, ygp1)


def kernel(x, expert_weights, top_k_indices, W, b):
    B, S, D = x.shape
    E, O, _ = W.shape
    K = expert_weights.shape[-1]
    T = B * S

    Wb = W.astype(jnp.bfloat16)

    row_token, row_w, tile_expert, pos0, pos1, J, NT = _routing_metadata(
        expert_weights, top_k_indices, T, K, E)

    x_bits = _tc_pack(x.reshape(T, D), T, D)
    xg_bits = _sc_row_gather(x_bits, row_token, J, D // 2)
    yg_bits = _tc_ragged_matmul(xg_bits, row_w, tile_expert, Wb, b,
                                J, NT, D, O, E)
    ygp0 = _sc_row_gather(yg_bits, pos0, T, O // 2)
    ygp1 = _sc_row_gather(yg_bits, pos1, T, O // 2)
    out = _tc_combine(ygp0, ygp1, T, O)
    return out.reshape(B, S, O)


# speedup vs baseline: 15.8135x; 1.1019x over previous
import functools

import jax
import jax.numpy as jnp
from jax import lax
from jax.experimental import pallas as pl
from jax.experimental.pallas import tpu as pltpu
from jax.experimental.pallas import tpu_sc as plsc

TMR = 256


def _routing_metadata(expert_weights, top_k_indices, T, K, E):
    P = T * K
    e_flat = top_k_indices.reshape(P).astype(jnp.int32)
    w_flat = expert_weights.reshape(P)
    onehot = (e_flat[:, None] == jnp.arange(E, dtype=jnp.int32)[None, :]).astype(jnp.int32)
    csum = jnp.cumsum(onehot, axis=0)
    counts = csum[-1]
    rank = jnp.take_along_axis(csum, e_flat[:, None], axis=1)[:, 0] - 1
    padded_counts = ((counts + TMR - 1) // TMR) * TMR
    cum_padded = jnp.cumsum(padded_counts)
    padded_offsets = cum_padded - padded_counts
    pos = padded_offsets[e_flat] + rank
    J = P + E * TMR
    NT = J // TMR
    row_token = jnp.zeros((J,), jnp.int32).at[pos].set(
        jnp.arange(P, dtype=jnp.int32) // K)
    tile_starts = jnp.arange(NT, dtype=jnp.int32) * TMR
    tile_expert = jnp.minimum(
        jnp.searchsorted(cum_padded, tile_starts, side="right").astype(jnp.int32),
        E - 1)
    pos2 = pos.reshape(T, K)
    pos_cat = jnp.concatenate([pos2[:, 0], pos2[:, 1]])
    return row_token, tile_expert, pos_cat, J, NT


def _pack_halves(lo_f32, hi_f32):
    lo_u = lax.bitcast_convert_type(lo_f32, jnp.uint32)
    hi_u = lax.bitcast_convert_type(hi_f32, jnp.uint32)
    lo_r = (lo_u + 0x8000) >> 16
    hi_r = (hi_u + 0x8000) & jnp.uint32(0xFFFF0000)
    return lax.bitcast_convert_type(lo_r | hi_r, jnp.int32)


def _unpack_halves(words_i32):
    u = lax.bitcast_convert_type(words_i32, jnp.uint32)
    lo = lax.bitcast_convert_type(u << 16, jnp.float32)
    hi = lax.bitcast_convert_type(u & jnp.uint32(0xFFFF0000), jnp.float32)
    return lo, hi


def _tc_pack_body(x_ref, o_ref):
    half = o_ref.shape[1]
    o_ref[...] = _pack_halves(x_ref[:, :half], x_ref[:, half:])


def _tc_pack(x2, T, D):
    TMP = 512
    return pl.pallas_call(
        _tc_pack_body,
        grid=(T // TMP,),
        in_specs=[pl.BlockSpec((TMP, D), lambda i: (i, 0))],
        out_specs=pl.BlockSpec((TMP, D // 2), lambda i: (i, 0)),
        out_shape=jax.ShapeDtypeStruct((T, D // 2), jnp.int32),
    )(x2)


def _sc_row_gather(src, indices, n_out, width):
    info = plsc.get_sparse_core_info()
    NC, NS = info.num_cores, info.num_subcores
    NW = NC * NS
    rows_per_w = n_out // NW
    CH = 32
    nchunk = rows_per_w // CH
    NBUF, H = 3, 2
    mesh = plsc.VectorSubcoreMesh(core_axis_name="c", subcore_axis_name="s")

    @functools.partial(
        pl.kernel, mesh=mesh,
        out_type=jax.ShapeDtypeStruct((n_out, width), jnp.int32),
        scratch_types=[
            pltpu.VMEM((nchunk, 1, CH), jnp.int32),
            pltpu.VMEM((NBUF, CH, width), jnp.int32),
        ] + [pltpu.SemaphoreType.DMA] * (2 * NBUF),
    )
    def gather_k(src_hbm, idx_hbm, out_hbm, idx_v, bufs, *sems):
        gsem = sems[:NBUF]
        ssem = sems[NBUF:]
        wid = lax.axis_index("s") * NC + lax.axis_index("c")
        base0 = wid * rows_per_w
        pltpu.sync_copy(idx_hbm.at[pl.ds(wid * nchunk, nchunk)], idx_v)

        def body(v, _):
            @pl.when(v < nchunk)
            def _():
                def start(b):
                    @pl.when(v >= NBUF)
                    def _():
                        pltpu.make_async_copy(
                            bufs.at[b], out_hbm.at[pl.ds(base0, CH)],
                            ssem[b]).wait()
                    pltpu.async_copy(
                        src_hbm.at[idx_v.at[v, 0]], bufs.at[b], gsem[b])
                _on_slot(lax.rem(v, NBUF), NBUF, start)

            @pl.when(v >= H)
            def _():
                cc = v - H

                def finish(b):
                    pltpu.make_async_copy(
                        src_hbm.at[idx_v.at[cc, 0]], bufs.at[b],
                        gsem[b]).wait()
                    pltpu.async_copy(
                        bufs.at[b], out_hbm.at[pl.ds(base0 + cc * CH, CH)],
                        ssem[b])
                _on_slot(lax.rem(cc, NBUF), NBUF, finish)
            return 0

        lax.fori_loop(0, nchunk + H, body, 0)
        for c in range(nchunk - NBUF, nchunk):
            pltpu.make_async_copy(
                bufs.at[c % NBUF], out_hbm.at[pl.ds(base0, CH)],
                ssem[c % NBUF]).wait()

    return gather_k(src.reshape(-1, width),
                    indices.reshape(NW * nchunk, 1, CH))


def _on_slot(slot, nbuf, fn):
    for b in range(nbuf):
        pl.when(slot == b)(functools.partial(fn, b))


def _tc_matmul_body(te_ref, xg_ref, w_ref, b_ref, o_ref):
    halfk = xg_ref.shape[1]
    lo, hi = _unpack_halves(xg_ref[...])
    w = w_ref[0]
    mm = lax.dot_general(lo.astype(jnp.bfloat16), w[:, :halfk],
                         (((1,), (1,)), ((), ())),
                         preferred_element_type=jnp.float32)
    mm = mm + lax.dot_general(hi.astype(jnp.bfloat16), w[:, halfk:],
                              (((1,), (1,)), ((), ())),
                              preferred_element_type=jnp.float32)
    y = mm + b_ref[0]
    halfo = o_ref.shape[1]
    o_ref[...] = _pack_halves(y[:, :halfo], y[:, halfo:])


def _tc_ragged_matmul(xg_bits, tile_expert, Wb, b, J, NT, D, O, E):
    grid_spec = pltpu.PrefetchScalarGridSpec(
        num_scalar_prefetch=1,
        grid=(NT,),
        in_specs=[
            pl.BlockSpec((TMR, D // 2), lambda i, te: (i, 0)),
            pl.BlockSpec((1, O, D), lambda i, te: (te[i], 0, 0)),
            pl.BlockSpec((1, 1, O), lambda i, te: (te[i], 0, 0)),
        ],
        out_specs=pl.BlockSpec((TMR, O // 2), lambda i, te: (i, 0)),
    )
    return pl.pallas_call(
        _tc_matmul_body,
        grid_spec=grid_spec,
        out_shape=jax.ShapeDtypeStruct((J, O // 2), jnp.int32),
    )(tile_expert, xg_bits, Wb, b.reshape(E, 1, O))


def _tc_combine_body(ew_ref, a_ref, b_ref, o_ref):
    alo, ahi = _unpack_halves(a_ref[...])
    blo, bhi = _unpack_halves(b_ref[...])
    w0 = ew_ref[:, 0:1]
    w1 = ew_ref[:, 1:2]
    half = a_ref.shape[1]
    o_ref[:, :half] = w0 * alo + w1 * blo
    o_ref[:, half:] = w0 * ahi + w1 * bhi


def _tc_combine(ygp, ew, T, O, K):
    TMC = 512
    nblk = T // TMC
    return pl.pallas_call(
        _tc_combine_body,
        grid=(nblk,),
        in_specs=[
            pl.BlockSpec((TMC, K), lambda i: (i, 0)),
            pl.BlockSpec((TMC, O // 2), lambda i: (i, 0)),
            pl.BlockSpec((TMC, O // 2), lambda i, n=nblk: (i + n, 0)),
        ],
        out_specs=pl.BlockSpec((TMC, O), lambda i: (i, 0)),
        out_shape=jax.ShapeDtypeStruct((T, O), jnp.float32),
    )(ew, ygp, ygp)


def kernel(x, expert_weights, top_k_indices, W, b):
    B, S, D = x.shape
    E, O, _ = W.shape
    K = expert_weights.shape[-1]
    T = B * S

    Wb = W.astype(jnp.bfloat16)

    row_token, tile_expert, pos_cat, J, NT = _routing_metadata(
        expert_weights, top_k_indices, T, K, E)

    x_bits = _tc_pack(x.reshape(T, D), T, D)
    xg_bits = _sc_row_gather(x_bits, row_token, J, D // 2)
    yg_bits = _tc_ragged_matmul(xg_bits, tile_expert, Wb, b, J, NT, D, O, E)
    ygp = _sc_row_gather(yg_bits, pos_cat, T * K, O // 2)
    out = _tc_combine(ygp, expert_weights.reshape(T, K), T, O, K)
    return out.reshape(B, S, O)


# speedup vs baseline: 21.2083x; 1.3411x over previous
import functools

import jax
import jax.numpy as jnp
from jax import lax
from jax.experimental import pallas as pl
from jax.experimental.pallas import tpu as pltpu
from jax.experimental.pallas import tpu_sc as plsc

TMR = 256


def _routing_metadata(expert_weights, top_k_indices, T, K, E):
    P = T * K
    e_flat = top_k_indices.reshape(P).astype(jnp.int32)
    w_flat = expert_weights.reshape(P)
    onehot = (e_flat[:, None] == jnp.arange(E, dtype=jnp.int32)[None, :]).astype(jnp.int32)
    csum = jnp.cumsum(onehot, axis=0)
    counts = csum[-1]
    rank = jnp.take_along_axis(csum, e_flat[:, None], axis=1)[:, 0] - 1
    padded_counts = ((counts + TMR - 1) // TMR) * TMR
    cum_padded = jnp.cumsum(padded_counts)
    padded_offsets = cum_padded - padded_counts
    pos = padded_offsets[e_flat] + rank
    J = P + E * TMR
    NT = J // TMR
    tile_starts = jnp.arange(NT, dtype=jnp.int32) * TMR
    tile_expert = jnp.minimum(
        jnp.searchsorted(cum_padded, tile_starts, side="right").astype(jnp.int32),
        E - 1)
    pos2 = pos.reshape(T, K)
    pos_cat = jnp.concatenate([pos2[:, 0], pos2[:, 1]])
    return pos, tile_expert, pos_cat, J, NT


def _pack_halves(lo_f32, hi_f32):
    lo_u = lax.bitcast_convert_type(lo_f32, jnp.uint32)
    hi_u = lax.bitcast_convert_type(hi_f32, jnp.uint32)
    lo_r = (lo_u + 0x8000) >> 16
    hi_r = (hi_u + 0x8000) & jnp.uint32(0xFFFF0000)
    return lax.bitcast_convert_type(lo_r | hi_r, jnp.int32)


def _unpack_halves(words_i32):
    u = lax.bitcast_convert_type(words_i32, jnp.uint32)
    lo = lax.bitcast_convert_type(u << 16, jnp.float32)
    hi = lax.bitcast_convert_type(u & jnp.uint32(0xFFFF0000), jnp.float32)
    return lo, hi


def _tc_pack_body(x_ref, o_ref):
    half = o_ref.shape[1]
    o_ref[...] = _pack_halves(x_ref[:, :half], x_ref[:, half:])


def _tc_pack(x2, T, D):
    TMP = 512
    return pl.pallas_call(
        _tc_pack_body,
        grid=(T // TMP,),
        in_specs=[pl.BlockSpec((TMP, D), lambda i: (i, 0))],
        out_specs=pl.BlockSpec((TMP, D // 2), lambda i: (i, 0)),
        out_shape=jax.ShapeDtypeStruct((T, D // 2), jnp.int32),
    )(x2)


def _sc_row_gather(src, indices, n_out, width):
    info = plsc.get_sparse_core_info()
    NC, NS = info.num_cores, info.num_subcores
    NW = NC * NS
    rows_per_w = n_out // NW
    CH = 32
    nchunk = rows_per_w // CH
    NBUF, H = 3, 2
    mesh = plsc.VectorSubcoreMesh(core_axis_name="c", subcore_axis_name="s")

    @functools.partial(
        pl.kernel, mesh=mesh,
        out_type=jax.ShapeDtypeStruct((n_out, width), jnp.int32),
        scratch_types=[
            pltpu.VMEM((nchunk, 1, CH), jnp.int32),
            pltpu.VMEM((NBUF, CH, width), jnp.int32),
        ] + [pltpu.SemaphoreType.DMA] * (2 * NBUF),
    )
    def gather_k(src_hbm, idx_hbm, out_hbm, idx_v, bufs, *sems):
        gsem = sems[:NBUF]
        ssem = sems[NBUF:]
        wid = lax.axis_index("s") * NC + lax.axis_index("c")
        base0 = wid * rows_per_w
        pltpu.sync_copy(idx_hbm.at[pl.ds(wid * nchunk, nchunk)], idx_v)

        def body(v, _):
            @pl.when(v < nchunk)
            def _():
                def start(b):
                    @pl.when(v >= NBUF)
                    def _():
                        pltpu.make_async_copy(
                            bufs.at[b], out_hbm.at[pl.ds(base0, CH)],
                            ssem[b]).wait()
                    pltpu.async_copy(
                        src_hbm.at[idx_v.at[v, 0]], bufs.at[b], gsem[b])
                _on_slot(lax.rem(v, NBUF), NBUF, start)

            @pl.when(v >= H)
            def _():
                cc = v - H

                def finish(b):
                    pltpu.make_async_copy(
                        src_hbm.at[idx_v.at[cc, 0]], bufs.at[b],
                        gsem[b]).wait()
                    pltpu.async_copy(
                        bufs.at[b], out_hbm.at[pl.ds(base0 + cc * CH, CH)],
                        ssem[b])
                _on_slot(lax.rem(cc, NBUF), NBUF, finish)
            return 0

        lax.fori_loop(0, nchunk + H, body, 0)
        for c in range(nchunk - NBUF, nchunk):
            pltpu.make_async_copy(
                bufs.at[c % NBUF], out_hbm.at[pl.ds(base0, CH)],
                ssem[c % NBUF]).wait()

    return gather_k(src.reshape(-1, width),
                    indices.reshape(NW * nchunk, 1, CH))


def _on_slot(slot, nbuf, fn):
    for b in range(nbuf):
        pl.when(slot == b)(functools.partial(fn, b))


def _sc_row_scatter(src, src_idx, dst_idx, n_items, n_out, width):
    info = plsc.get_sparse_core_info()
    NC, NS = info.num_cores, info.num_subcores
    NW = NC * NS
    rows_per_w = n_items // NW
    CH = 32
    nchunk = rows_per_w // CH
    NBUF, H = 3, 2
    mesh = plsc.VectorSubcoreMesh(core_axis_name="c", subcore_axis_name="s")

    @functools.partial(
        pl.kernel, mesh=mesh,
        out_type=jax.ShapeDtypeStruct((n_out, width), jnp.int32),
        scratch_types=[
            pltpu.VMEM((nchunk, 1, CH), jnp.int32),
            pltpu.VMEM((nchunk, 1, CH), jnp.int32),
            pltpu.VMEM((NBUF, CH, width), jnp.int32),
        ] + [pltpu.SemaphoreType.DMA] * (2 * NBUF),
    )
    def scatter_k(src_hbm, gidx_hbm, sidx_hbm, out_hbm, gidx_v, sidx_v,
                  bufs, *sems):
        gsem = sems[:NBUF]
        ssem = sems[NBUF:]
        wid = lax.axis_index("s") * NC + lax.axis_index("c")
        base0 = wid * rows_per_w
        pltpu.sync_copy(gidx_hbm.at[pl.ds(wid * nchunk, nchunk)], gidx_v)
        pltpu.sync_copy(sidx_hbm.at[pl.ds(wid * nchunk, nchunk)], sidx_v)

        def body(v, _):
            @pl.when(v < nchunk)
            def _():
                def start(b):
                    @pl.when(v >= NBUF)
                    def _():
                        pltpu.make_async_copy(
                            bufs.at[b], out_hbm.at[pl.ds(base0, CH)],
                            ssem[b]).wait()
                    pltpu.async_copy(
                        src_hbm.at[gidx_v.at[v, 0]], bufs.at[b], gsem[b])
                _on_slot(lax.rem(v, NBUF), NBUF, start)

            @pl.when(v >= H)
            def _():
                cc = v - H

                def finish(b):
                    pltpu.make_async_copy(
                        src_hbm.at[gidx_v.at[cc, 0]], bufs.at[b],
                        gsem[b]).wait()
                    pltpu.async_copy(
                        bufs.at[b], out_hbm.at[sidx_v.at[cc, 0]], ssem[b])
                _on_slot(lax.rem(cc, NBUF), NBUF, finish)
            return 0

        lax.fori_loop(0, nchunk + H, body, 0)
        for c in range(nchunk - NBUF, nchunk):
            pltpu.make_async_copy(
                bufs.at[c % NBUF], out_hbm.at[pl.ds(base0, CH)],
                ssem[c % NBUF]).wait()

    return scatter_k(src.reshape(-1, width),
                     src_idx.reshape(NW * nchunk, 1, CH),
                     dst_idx.reshape(NW * nchunk, 1, CH))


def _tc_matmul_body(te_ref, xg_ref, w_ref, b_ref, o_ref):
    halfk = xg_ref.shape[1]
    lo, hi = _unpack_halves(xg_ref[...])
    w = w_ref[0]
    mm = lax.dot_general(lo.astype(jnp.bfloat16), w[:, :halfk],
                         (((1,), (1,)), ((), ())),
                         preferred_element_type=jnp.float32)
    mm = mm + lax.dot_general(hi.astype(jnp.bfloat16), w[:, halfk:],
                              (((1,), (1,)), ((), ())),
                              preferred_element_type=jnp.float32)
    y = mm + b_ref[0]
    halfo = o_ref.shape[1]
    o_ref[...] = _pack_halves(y[:, :halfo], y[:, halfo:])


def _tc_ragged_matmul(xg_bits, tile_expert, Wb, b, J, NT, D, O, E):
    grid_spec = pltpu.PrefetchScalarGridSpec(
        num_scalar_prefetch=1,
        grid=(NT,),
        in_specs=[
            pl.BlockSpec((TMR, D // 2), lambda i, te: (i, 0)),
            pl.BlockSpec((1, O, D), lambda i, te: (te[i], 0, 0)),
            pl.BlockSpec((1, 1, O), lambda i, te: (te[i], 0, 0)),
        ],
        out_specs=pl.BlockSpec((TMR, O // 2), lambda i, te: (i, 0)),
    )
    return pl.pallas_call(
        _tc_matmul_body,
        grid_spec=grid_spec,
        out_shape=jax.ShapeDtypeStruct((J, O // 2), jnp.int32),
    )(tile_expert, xg_bits, Wb, b.reshape(E, 1, O))


def _tc_combine_body(ew_ref, a_ref, b_ref, o_ref):
    alo, ahi = _unpack_halves(a_ref[...])
    blo, bhi = _unpack_halves(b_ref[...])
    w0 = ew_ref[:, 0:1]
    w1 = ew_ref[:, 1:2]
    half = a_ref.shape[1]
    o_ref[:, :half] = w0 * alo + w1 * blo
    o_ref[:, half:] = w0 * ahi + w1 * bhi


def _tc_combine(ygp, ew, T, O, K):
    TMC = 512
    nblk = T // TMC
    return pl.pallas_call(
        _tc_combine_body,
        grid=(nblk,),
        in_specs=[
            pl.BlockSpec((TMC, K), lambda i: (i, 0)),
            pl.BlockSpec((TMC, O // 2), lambda i: (i, 0)),
            pl.BlockSpec((TMC, O // 2), lambda i, n=nblk: (i + n, 0)),
        ],
        out_specs=pl.BlockSpec((TMC, O), lambda i: (i, 0)),
        out_shape=jax.ShapeDtypeStruct((T, O), jnp.float32),
    )(ew, ygp, ygp)


def kernel(x, expert_weights, top_k_indices, W, b):
    B, S, D = x.shape
    E, O, _ = W.shape
    K = expert_weights.shape[-1]
    T = B * S

    Wb = W.astype(jnp.bfloat16)

    pos, tile_expert, pos_cat, J, NT = _routing_metadata(
        expert_weights, top_k_indices, T, K, E)

    x_bits = _tc_pack(x.reshape(T, D), T, D)
    gidx = jnp.arange(T * K, dtype=jnp.int32) // K
    xg_bits = _sc_row_scatter(x_bits, gidx, pos, T * K, J, D // 2)
    yg_bits = _tc_ragged_matmul(xg_bits, tile_expert, Wb, b, J, NT, D, O, E)
    ygp = _sc_row_gather(yg_bits, pos_cat, T * K, O // 2)
    out = _tc_combine(ygp, expert_weights.reshape(T, K), T, O, K)
    return out.reshape(B, S, O)


# speedup vs baseline: 24.1054x; 1.1366x over previous
import functools

import jax
import jax.numpy as jnp
from jax import lax
from jax.experimental import pallas as pl
from jax.experimental.pallas import tpu as pltpu
from jax.experimental.pallas import tpu_sc as plsc

TMR = 256


def _routing_metadata(expert_weights, top_k_indices, T, K, E):
    P = T * K
    e_flat = top_k_indices.reshape(P).astype(jnp.int32)
    w_flat = expert_weights.reshape(P)
    onehot = (e_flat[:, None] == jnp.arange(E, dtype=jnp.int32)[None, :]).astype(jnp.int32)
    csum = jnp.cumsum(onehot, axis=0)
    counts = csum[-1]
    rank = jnp.take_along_axis(csum, e_flat[:, None], axis=1)[:, 0] - 1
    padded_counts = ((counts + TMR - 1) // TMR) * TMR
    cum_padded = jnp.cumsum(padded_counts)
    padded_offsets = cum_padded - padded_counts
    pos = padded_offsets[e_flat] + rank
    J = P + E * TMR
    NT = J // TMR
    tile_starts = jnp.arange(NT, dtype=jnp.int32) * TMR
    tile_expert = jnp.minimum(
        jnp.searchsorted(cum_padded, tile_starts, side="right").astype(jnp.int32),
        E - 1)
    pos2 = pos.reshape(T, K)
    pos_cat = jnp.concatenate([pos2[:, 0], pos2[:, 1]])
    return pos, tile_expert, pos_cat, J, NT


def _pack_halves(lo_f32, hi_f32):
    lo_u = lax.bitcast_convert_type(lo_f32, jnp.uint32)
    hi_u = lax.bitcast_convert_type(hi_f32, jnp.uint32)
    lo_r = (lo_u + 0x8000) >> 16
    hi_r = (hi_u + 0x8000) & jnp.uint32(0xFFFF0000)
    return lax.bitcast_convert_type(lo_r | hi_r, jnp.int32)


def _unpack_halves(words_i32):
    u = lax.bitcast_convert_type(words_i32, jnp.uint32)
    lo = lax.bitcast_convert_type(u << 16, jnp.float32)
    hi = lax.bitcast_convert_type(u & jnp.uint32(0xFFFF0000), jnp.float32)
    return lo, hi


def _tc_pack_body(x_ref, o_ref):
    half = o_ref.shape[1]
    o_ref[...] = _pack_halves(x_ref[:, :half], x_ref[:, half:])


def _tc_pack(x2, T, D):
    TMP = 512
    return pl.pallas_call(
        _tc_pack_body,
        grid=(T // TMP,),
        in_specs=[pl.BlockSpec((TMP, D), lambda i: (i, 0))],
        out_specs=pl.BlockSpec((TMP, D // 2), lambda i: (i, 0)),
        out_shape=jax.ShapeDtypeStruct((T, D // 2), jnp.int32),
    )(x2)


def _sc_row_gather(src, indices, n_out, width):
    info = plsc.get_sparse_core_info()
    NC, NS = info.num_cores, info.num_subcores
    NW = NC * NS
    rows_per_w = n_out // NW
    CH = 32
    nchunk = rows_per_w // CH
    NBUF, H = 3, 2
    mesh = plsc.VectorSubcoreMesh(core_axis_name="c", subcore_axis_name="s")

    @functools.partial(
        pl.kernel, mesh=mesh,
        out_type=jax.ShapeDtypeStruct((n_out, width), jnp.int32),
        scratch_types=[
            pltpu.VMEM((nchunk, 1, CH), jnp.int32),
            pltpu.VMEM((NBUF, CH, width), jnp.int32),
        ] + [pltpu.SemaphoreType.DMA] * (2 * NBUF),
    )
    def gather_k(src_hbm, idx_hbm, out_hbm, idx_v, bufs, *sems):
        gsem = sems[:NBUF]
        ssem = sems[NBUF:]
        wid = lax.axis_index("s") * NC + lax.axis_index("c")
        base0 = wid * rows_per_w
        pltpu.sync_copy(idx_hbm.at[pl.ds(wid * nchunk, nchunk)], idx_v)

        def body(v, _):
            @pl.when(v < nchunk)
            def _():
                def start(b):
                    @pl.when(v >= NBUF)
                    def _():
                        pltpu.make_async_copy(
                            bufs.at[b], out_hbm.at[pl.ds(base0, CH)],
                            ssem[b]).wait()
                    pltpu.async_copy(
                        src_hbm.at[idx_v.at[v, 0]], bufs.at[b], gsem[b])
                _on_slot(lax.rem(v, NBUF), NBUF, start)

            @pl.when(v >= H)
            def _():
                cc = v - H

                def finish(b):
                    pltpu.make_async_copy(
                        src_hbm.at[idx_v.at[cc, 0]], bufs.at[b],
                        gsem[b]).wait()
                    pltpu.async_copy(
                        bufs.at[b], out_hbm.at[pl.ds(base0 + cc * CH, CH)],
                        ssem[b])
                _on_slot(lax.rem(cc, NBUF), NBUF, finish)
            return 0

        lax.fori_loop(0, nchunk + H, body, 0)
        for c in range(nchunk - NBUF, nchunk):
            pltpu.make_async_copy(
                bufs.at[c % NBUF], out_hbm.at[pl.ds(base0, CH)],
                ssem[c % NBUF]).wait()

    return gather_k(src.reshape(-1, width),
                    indices.reshape(NW * nchunk, 1, CH))


def _on_slot(slot, nbuf, fn):
    for b in range(nbuf):
        pl.when(slot == b)(functools.partial(fn, b))


def _sc_row_scatter(src, src_idx, dst_idx, n_items, n_out, width):
    info = plsc.get_sparse_core_info()
    NC, NS = info.num_cores, info.num_subcores
    NW = NC * NS
    rows_per_w = n_items // NW
    CH = 32
    nchunk = rows_per_w // CH
    NBUF, H = 3, 2
    mesh = plsc.VectorSubcoreMesh(core_axis_name="c", subcore_axis_name="s")

    @functools.partial(
        pl.kernel, mesh=mesh,
        out_type=jax.ShapeDtypeStruct((n_out, width), jnp.int32),
        scratch_types=[
            pltpu.VMEM((nchunk, 1, CH), jnp.int32),
            pltpu.VMEM((nchunk, 1, CH), jnp.int32),
            pltpu.VMEM((NBUF, CH, width), jnp.int32),
        ] + [pltpu.SemaphoreType.DMA] * (2 * NBUF),
    )
    def scatter_k(src_hbm, gidx_hbm, sidx_hbm, out_hbm, gidx_v, sidx_v,
                  bufs, *sems):
        gsem = sems[:NBUF]
        ssem = sems[NBUF:]
        wid = lax.axis_index("s") * NC + lax.axis_index("c")
        base0 = wid * rows_per_w
        pltpu.sync_copy(gidx_hbm.at[pl.ds(wid * nchunk, nchunk)], gidx_v)
        pltpu.sync_copy(sidx_hbm.at[pl.ds(wid * nchunk, nchunk)], sidx_v)

        def body(v, _):
            @pl.when(v < nchunk)
            def _():
                def start(b):
                    @pl.when(v >= NBUF)
                    def _():
                        pltpu.make_async_copy(
                            bufs.at[b], out_hbm.at[pl.ds(base0, CH)],
                            ssem[b]).wait()
                    pltpu.async_copy(
                        src_hbm.at[gidx_v.at[v, 0]], bufs.at[b], gsem[b])
                _on_slot(lax.rem(v, NBUF), NBUF, start)

            @pl.when(v >= H)
            def _():
                cc = v - H

                def finish(b):
                    pltpu.make_async_copy(
                        src_hbm.at[gidx_v.at[cc, 0]], bufs.at[b],
                        gsem[b]).wait()
                    pltpu.async_copy(
                        bufs.at[b], out_hbm.at[sidx_v.at[cc, 0]], ssem[b])
                _on_slot(lax.rem(cc, NBUF), NBUF, finish)
            return 0

        lax.fori_loop(0, nchunk + H, body, 0)
        for c in range(nchunk - NBUF, nchunk):
            pltpu.make_async_copy(
                bufs.at[c % NBUF], out_hbm.at[pl.ds(base0, CH)],
                ssem[c % NBUF]).wait()

    return scatter_k(src.reshape(-1, width),
                     src_idx.reshape(NW * nchunk, 1, CH),
                     dst_idx.reshape(NW * nchunk, 1, CH))


def _tc_matmul_body(te_ref, xg_ref, w_ref, b_ref, o_ref):
    halfk = xg_ref.shape[1]
    lo, hi = _unpack_halves(xg_ref[...])
    w = w_ref[0]
    mm = lax.dot_general(lo, w[:, :halfk],
                         (((1,), (1,)), ((), ())),
                         preferred_element_type=jnp.float32,
                         precision=lax.Precision.DEFAULT)
    mm = mm + lax.dot_general(hi, w[:, halfk:],
                              (((1,), (1,)), ((), ())),
                              preferred_element_type=jnp.float32,
                              precision=lax.Precision.DEFAULT)
    y = mm + b_ref[0]
    halfo = o_ref.shape[1]
    o_ref[...] = _pack_halves(y[:, :halfo], y[:, halfo:])


def _tc_ragged_matmul(xg_bits, tile_expert, Wb, b, J, NT, D, O, E):
    grid_spec = pltpu.PrefetchScalarGridSpec(
        num_scalar_prefetch=1,
        grid=(NT,),
        in_specs=[
            pl.BlockSpec((TMR, D // 2), lambda i, te: (i, 0)),
            pl.BlockSpec((1, O, D), lambda i, te: (te[i], 0, 0)),
            pl.BlockSpec((1, 1, O), lambda i, te: (te[i], 0, 0)),
        ],
        out_specs=pl.BlockSpec((TMR, O // 2), lambda i, te: (i, 0)),
    )
    return pl.pallas_call(
        _tc_matmul_body,
        grid_spec=grid_spec,
        out_shape=jax.ShapeDtypeStruct((J, O // 2), jnp.int32),
    )(tile_expert, xg_bits, Wb, b.reshape(E, 1, O))


def _tc_combine_body(ew_ref, a_ref, b_ref, o_ref):
    alo, ahi = _unpack_halves(a_ref[...])
    blo, bhi = _unpack_halves(b_ref[...])
    w0 = ew_ref[:, 0:1]
    w1 = ew_ref[:, 1:2]
    half = a_ref.shape[1]
    o_ref[:, :half] = w0 * alo + w1 * blo
    o_ref[:, half:] = w0 * ahi + w1 * bhi


def _tc_combine(ygp, ew, T, O, K):
    TMC = 512
    nblk = T // TMC
    return pl.pallas_call(
        _tc_combine_body,
        grid=(nblk,),
        in_specs=[
            pl.BlockSpec((TMC, K), lambda i: (i, 0)),
            pl.BlockSpec((TMC, O // 2), lambda i: (i, 0)),
            pl.BlockSpec((TMC, O // 2), lambda i, n=nblk: (i + n, 0)),
        ],
        out_specs=pl.BlockSpec((TMC, O), lambda i: (i, 0)),
        out_shape=jax.ShapeDtypeStruct((T, O), jnp.float32),
    )(ew, ygp, ygp)


def kernel(x, expert_weights, top_k_indices, W, b):
    B, S, D = x.shape
    E, O, _ = W.shape
    K = expert_weights.shape[-1]
    T = B * S

    pos, tile_expert, pos_cat, J, NT = _routing_metadata(
        expert_weights, top_k_indices, T, K, E)

    x_bits = _tc_pack(x.reshape(T, D), T, D)
    gidx = jnp.arange(T * K, dtype=jnp.int32) // K
    xg_bits = _sc_row_scatter(x_bits, gidx, pos, T * K, J, D // 2)
    yg_bits = _tc_ragged_matmul(xg_bits, tile_expert, W, b, J, NT, D, O, E)
    ygp = _sc_row_gather(yg_bits, pos_cat, T * K, O // 2)
    out = _tc_combine(ygp, expert_weights.reshape(T, K), T, O, K)
    return out.reshape(B, S, O)


# speedup vs baseline: 25.4508x; 1.0558x over previous
import functools

import jax
import jax.numpy as jnp
from jax import lax
from jax.experimental import pallas as pl
from jax.experimental.pallas import tpu as pltpu
from jax.experimental.pallas import tpu_sc as plsc

TMR = 256


def _routing_metadata(expert_weights, top_k_indices, T, K, E):
    P = T * K
    e_flat = top_k_indices.reshape(P).astype(jnp.int32)
    w_flat = expert_weights.reshape(P)
    onehot = (e_flat[:, None] == jnp.arange(E, dtype=jnp.int32)[None, :]).astype(jnp.int32)
    csum = jnp.cumsum(onehot, axis=0)
    counts = csum[-1]
    rank = jnp.sum(onehot * csum, axis=1) - 1
    padded_counts = ((counts + TMR - 1) // TMR) * TMR
    cum_padded = jnp.cumsum(padded_counts)
    padded_offsets = cum_padded - padded_counts
    pos = padded_offsets[e_flat] + rank
    J = P + E * TMR
    NT = J // TMR
    tile_starts = jnp.arange(NT, dtype=jnp.int32) * TMR
    tile_expert = jnp.minimum(
        jnp.sum((tile_starts[:, None] >= cum_padded[None, :]).astype(jnp.int32),
                axis=1),
        E - 1)
    pos2 = pos.reshape(T, K)
    pos_cat = jnp.concatenate([pos2[:, 0], pos2[:, 1]])
    return pos, tile_expert, pos_cat, J, NT


def _pack_halves(lo_f32, hi_f32):
    lo_u = lax.bitcast_convert_type(lo_f32, jnp.uint32)
    hi_u = lax.bitcast_convert_type(hi_f32, jnp.uint32)
    lo_r = (lo_u + 0x8000) >> 16
    hi_r = (hi_u + 0x8000) & jnp.uint32(0xFFFF0000)
    return lax.bitcast_convert_type(lo_r | hi_r, jnp.int32)


def _unpack_halves(words_i32):
    u = lax.bitcast_convert_type(words_i32, jnp.uint32)
    lo = lax.bitcast_convert_type(u << 16, jnp.float32)
    hi = lax.bitcast_convert_type(u & jnp.uint32(0xFFFF0000), jnp.float32)
    return lo, hi


def _tc_pack_body(x_ref, o_ref):
    half = o_ref.shape[1]
    o_ref[...] = _pack_halves(x_ref[:, :half], x_ref[:, half:])


def _tc_pack(x2, T, D):
    TMP = 512
    return pl.pallas_call(
        _tc_pack_body,
        grid=(T // TMP,),
        in_specs=[pl.BlockSpec((TMP, D), lambda i: (i, 0))],
        out_specs=pl.BlockSpec((TMP, D // 2), lambda i: (i, 0)),
        out_shape=jax.ShapeDtypeStruct((T, D // 2), jnp.int32),
    )(x2)


def _sc_row_gather(src, indices, n_out, width):
    info = plsc.get_sparse_core_info()
    NC, NS = info.num_cores, info.num_subcores
    NW = NC * NS
    rows_per_w = n_out // NW
    CH = 32
    nchunk = rows_per_w // CH
    NBUF, H = 3, 2
    mesh = plsc.VectorSubcoreMesh(core_axis_name="c", subcore_axis_name="s")

    @functools.partial(
        pl.kernel, mesh=mesh,
        out_type=jax.ShapeDtypeStruct((n_out, width), jnp.int32),
        scratch_types=[
            pltpu.VMEM((nchunk, 1, CH), jnp.int32),
            pltpu.VMEM((NBUF, CH, width), jnp.int32),
        ] + [pltpu.SemaphoreType.DMA] * (2 * NBUF),
    )
    def gather_k(src_hbm, idx_hbm, out_hbm, idx_v, bufs, *sems):
        gsem = sems[:NBUF]
        ssem = sems[NBUF:]
        wid = lax.axis_index("s") * NC + lax.axis_index("c")
        base0 = wid * rows_per_w
        pltpu.sync_copy(idx_hbm.at[pl.ds(wid * nchunk, nchunk)], idx_v)

        def body(v, _):
            @pl.when(v < nchunk)
            def _():
                def start(b):
                    @pl.when(v >= NBUF)
                    def _():
                        pltpu.make_async_copy(
                            bufs.at[b], out_hbm.at[pl.ds(base0, CH)],
                            ssem[b]).wait()
                    pltpu.async_copy(
                        src_hbm.at[idx_v.at[v, 0]], bufs.at[b], gsem[b])
                _on_slot(lax.rem(v, NBUF), NBUF, start)

            @pl.when(v >= H)
            def _():
                cc = v - H

                def finish(b):
                    pltpu.make_async_copy(
                        src_hbm.at[idx_v.at[cc, 0]], bufs.at[b],
                        gsem[b]).wait()
                    pltpu.async_copy(
                        bufs.at[b], out_hbm.at[pl.ds(base0 + cc * CH, CH)],
                        ssem[b])
                _on_slot(lax.rem(cc, NBUF), NBUF, finish)
            return 0

        lax.fori_loop(0, nchunk + H, body, 0)
        for c in range(nchunk - NBUF, nchunk):
            pltpu.make_async_copy(
                bufs.at[c % NBUF], out_hbm.at[pl.ds(base0, CH)],
                ssem[c % NBUF]).wait()

    return gather_k(src.reshape(-1, width),
                    indices.reshape(NW * nchunk, 1, CH))


def _on_slot(slot, nbuf, fn):
    for b in range(nbuf):
        pl.when(slot == b)(functools.partial(fn, b))


def _sc_row_scatter(src, src_idx, dst_idx, n_items, n_out, width):
    info = plsc.get_sparse_core_info()
    NC, NS = info.num_cores, info.num_subcores
    NW = NC * NS
    rows_per_w = n_items // NW
    CH = 32
    nchunk = rows_per_w // CH
    NBUF, H = 3, 2
    mesh = plsc.VectorSubcoreMesh(core_axis_name="c", subcore_axis_name="s")

    @functools.partial(
        pl.kernel, mesh=mesh,
        out_type=jax.ShapeDtypeStruct((n_out, width), jnp.int32),
        scratch_types=[
            pltpu.VMEM((nchunk, 1, CH), jnp.int32),
            pltpu.VMEM((nchunk, 1, CH), jnp.int32),
            pltpu.VMEM((NBUF, CH, width), jnp.int32),
        ] + [pltpu.SemaphoreType.DMA] * (2 * NBUF),
    )
    def scatter_k(src_hbm, gidx_hbm, sidx_hbm, out_hbm, gidx_v, sidx_v,
                  bufs, *sems):
        gsem = sems[:NBUF]
        ssem = sems[NBUF:]
        wid = lax.axis_index("s") * NC + lax.axis_index("c")
        base0 = wid * rows_per_w
        pltpu.sync_copy(gidx_hbm.at[pl.ds(wid * nchunk, nchunk)], gidx_v)
        pltpu.sync_copy(sidx_hbm.at[pl.ds(wid * nchunk, nchunk)], sidx_v)

        def body(v, _):
            @pl.when(v < nchunk)
            def _():
                def start(b):
                    @pl.when(v >= NBUF)
                    def _():
                        pltpu.make_async_copy(
                            bufs.at[b], out_hbm.at[pl.ds(base0, CH)],
                            ssem[b]).wait()
                    pltpu.async_copy(
                        src_hbm.at[gidx_v.at[v, 0]], bufs.at[b], gsem[b])
                _on_slot(lax.rem(v, NBUF), NBUF, start)

            @pl.when(v >= H)
            def _():
                cc = v - H

                def finish(b):
                    pltpu.make_async_copy(
                        src_hbm.at[gidx_v.at[cc, 0]], bufs.at[b],
                        gsem[b]).wait()
                    pltpu.async_copy(
                        bufs.at[b], out_hbm.at[sidx_v.at[cc, 0]], ssem[b])
                _on_slot(lax.rem(cc, NBUF), NBUF, finish)
            return 0

        lax.fori_loop(0, nchunk + H, body, 0)
        for c in range(nchunk - NBUF, nchunk):
            pltpu.make_async_copy(
                bufs.at[c % NBUF], out_hbm.at[pl.ds(base0, CH)],
                ssem[c % NBUF]).wait()

    return scatter_k(src.reshape(-1, width),
                     src_idx.reshape(NW * nchunk, 1, CH),
                     dst_idx.reshape(NW * nchunk, 1, CH))


def _tc_matmul_body(te_ref, xg_ref, w_ref, b_ref, o_ref):
    halfk = xg_ref.shape[1]
    lo, hi = _unpack_halves(xg_ref[...])
    w = w_ref[0]
    mm = lax.dot_general(lo, w[:, :halfk],
                         (((1,), (1,)), ((), ())),
                         preferred_element_type=jnp.float32,
                         precision=lax.Precision.DEFAULT)
    mm = mm + lax.dot_general(hi, w[:, halfk:],
                              (((1,), (1,)), ((), ())),
                              preferred_element_type=jnp.float32,
                              precision=lax.Precision.DEFAULT)
    y = mm + b_ref[0]
    halfo = o_ref.shape[1]
    o_ref[...] = _pack_halves(y[:, :halfo], y[:, halfo:])


def _tc_ragged_matmul(xg_bits, tile_expert, Wb, b, J, NT, D, O, E):
    grid_spec = pltpu.PrefetchScalarGridSpec(
        num_scalar_prefetch=1,
        grid=(NT,),
        in_specs=[
            pl.BlockSpec((TMR, D // 2), lambda i, te: (i, 0)),
            pl.BlockSpec((1, O, D), lambda i, te: (te[i], 0, 0)),
            pl.BlockSpec((1, 1, O), lambda i, te: (te[i], 0, 0)),
        ],
        out_specs=pl.BlockSpec((TMR, O // 2), lambda i, te: (i, 0)),
    )
    return pl.pallas_call(
        _tc_matmul_body,
        grid_spec=grid_spec,
        out_shape=jax.ShapeDtypeStruct((J, O // 2), jnp.int32),
    )(tile_expert, xg_bits, Wb, b.reshape(E, 1, O))


def _tc_combine_body(ew_ref, a_ref, b_ref, o_ref):
    alo, ahi = _unpack_halves(a_ref[...])
    blo, bhi = _unpack_halves(b_ref[...])
    w0 = ew_ref[:, 0:1]
    w1 = ew_ref[:, 1:2]
    half = a_ref.shape[1]
    o_ref[:, :half] = w0 * alo + w1 * blo
    o_ref[:, half:] = w0 * ahi + w1 * bhi


def _tc_combine(ygp, ew, T, O, K):
    TMC = 512
    nblk = T // TMC
    return pl.pallas_call(
        _tc_combine_body,
        grid=(nblk,),
        in_specs=[
            pl.BlockSpec((TMC, K), lambda i: (i, 0)),
            pl.BlockSpec((TMC, O // 2), lambda i: (i, 0)),
            pl.BlockSpec((TMC, O // 2), lambda i, n=nblk: (i + n, 0)),
        ],
        out_specs=pl.BlockSpec((TMC, O), lambda i: (i, 0)),
        out_shape=jax.ShapeDtypeStruct((T, O), jnp.float32),
    )(ew, ygp, ygp)


def kernel(x, expert_weights, top_k_indices, W, b):
    B, S, D = x.shape
    E, O, _ = W.shape
    K = expert_weights.shape[-1]
    T = B * S

    pos, tile_expert, pos_cat, J, NT = _routing_metadata(
        expert_weights, top_k_indices, T, K, E)

    x_bits = _tc_pack(x.reshape(T, D), T, D)
    gidx = jnp.arange(T * K, dtype=jnp.int32) // K
    xg_bits = _sc_row_scatter(x_bits, gidx, pos, T * K, J, D // 2)
    yg_bits = _tc_ragged_matmul(xg_bits, tile_expert, W, b, J, NT, D, O, E)
    ygp = _sc_row_gather(yg_bits, pos_cat, T * K, O // 2)
    out = _tc_combine(ygp, expert_weights.reshape(T, K), T, O, K)
    return out.reshape(B, S, O)


# speedup vs baseline: 25.4575x; 1.0003x over previous
import functools

import jax
import jax.numpy as jnp
from jax import lax
from jax.experimental import pallas as pl
from jax.experimental.pallas import tpu as pltpu
from jax.experimental.pallas import tpu_sc as plsc

TMR = 256


def _routing_metadata(expert_weights, top_k_indices, T, K, E):
    P = T * K
    e_flat = top_k_indices.reshape(P).astype(jnp.int32)
    onehot = (e_flat[:, None] == jnp.arange(E, dtype=jnp.int32)[None, :]).astype(jnp.int32)
    csum = jnp.cumsum(onehot, axis=0)
    counts = csum[-1]
    rank = jnp.sum(onehot * csum, axis=1) - 1
    padded_counts = ((counts + TMR - 1) // TMR) * TMR
    cum_padded = jnp.cumsum(padded_counts)
    padded_offsets = cum_padded - padded_counts
    pos = padded_offsets[e_flat] + rank
    J = P + E * TMR
    NT = J // TMR
    tile_starts = jnp.arange(NT, dtype=jnp.int32) * TMR
    tile_expert = jnp.minimum(
        jnp.sum((tile_starts[:, None] >= cum_padded[None, :]).astype(jnp.int32),
                axis=1),
        E - 1)
    pos2 = pos.reshape(T, K)
    pos_cat = jnp.concatenate([pos2[:, 0], pos2[:, 1]])
    return pos, tile_expert, pos_cat, J, NT


def _pack_halves(lo_f32, hi_f32):
    lo_u = lax.bitcast_convert_type(lo_f32, jnp.uint32)
    hi_u = lax.bitcast_convert_type(hi_f32, jnp.uint32)
    lo_r = (lo_u + 0x8000) >> 16
    hi_r = (hi_u + 0x8000) & jnp.uint32(0xFFFF0000)
    return lax.bitcast_convert_type(lo_r | hi_r, jnp.int32)


def _unpack_halves(words_i32):
    u = lax.bitcast_convert_type(words_i32, jnp.uint32)
    lo = lax.bitcast_convert_type(u << 16, jnp.float32)
    hi = lax.bitcast_convert_type(u & jnp.uint32(0xFFFF0000), jnp.float32)
    return lo, hi


def _tc_pack_body(x_ref, o_ref):
    half = o_ref.shape[1]
    o_ref[...] = _pack_halves(x_ref[:, :half], x_ref[:, half:])


def _tc_pack(x2, T, D):
    TMP = 512
    return pl.pallas_call(
        _tc_pack_body,
        grid=(T // TMP,),
        in_specs=[pl.BlockSpec((TMP, D), lambda i: (i, 0))],
        out_specs=pl.BlockSpec((TMP, D // 2), lambda i: (i, 0)),
        out_shape=jax.ShapeDtypeStruct((T, D // 2), jnp.int32),
    )(x2)


def _sc_row_gather(src, indices, n_out, width):
    info = plsc.get_sparse_core_info()
    NC, NS = info.num_cores, info.num_subcores
    NW = NC * NS
    rows_per_w = n_out // NW
    CH = 32
    nchunk = rows_per_w // CH
    NBUF, H = 3, 2
    mesh = plsc.VectorSubcoreMesh(core_axis_name="c", subcore_axis_name="s")

    @functools.partial(
        pl.kernel, mesh=mesh,
        out_type=jax.ShapeDtypeStruct((n_out, width), jnp.int32),
        scratch_types=[
            pltpu.VMEM((nchunk, 1, CH), jnp.int32),
            pltpu.VMEM((NBUF, CH, width), jnp.int32),
        ] + [pltpu.SemaphoreType.DMA] * (2 * NBUF),
    )
    def gather_k(src_hbm, idx_hbm, out_hbm, idx_v, bufs, *sems):
        gsem = sems[:NBUF]
        ssem = sems[NBUF:]
        wid = lax.axis_index("s") * NC + lax.axis_index("c")
        base0 = wid * rows_per_w
        pltpu.sync_copy(idx_hbm.at[pl.ds(wid * nchunk, nchunk)], idx_v)

        def body(v, _):
            @pl.when(v < nchunk)
            def _():
                def start(b):
                    @pl.when(v >= NBUF)
                    def _():
                        pltpu.make_async_copy(
                            bufs.at[b], out_hbm.at[pl.ds(base0, CH)],
                            ssem[b]).wait()
                    pltpu.async_copy(
                        src_hbm.at[idx_v.at[v, 0]], bufs.at[b], gsem[b])
                _on_slot(lax.rem(v, NBUF), NBUF, start)

            @pl.when(v >= H)
            def _():
                cc = v - H

                def finish(b):
                    pltpu.make_async_copy(
                        src_hbm.at[idx_v.at[cc, 0]], bufs.at[b],
                        gsem[b]).wait()
                    pltpu.async_copy(
                        bufs.at[b], out_hbm.at[pl.ds(base0 + cc * CH, CH)],
                        ssem[b])
                _on_slot(lax.rem(cc, NBUF), NBUF, finish)
            return 0

        lax.fori_loop(0, nchunk + H, body, 0)
        for c in range(nchunk - NBUF, nchunk):
            pltpu.make_async_copy(
                bufs.at[c % NBUF], out_hbm.at[pl.ds(base0, CH)],
                ssem[c % NBUF]).wait()

    return gather_k(src.reshape(-1, width),
                    indices.reshape(NW * nchunk, 1, CH))


def _on_slot(slot, nbuf, fn):
    for b in range(nbuf):
        pl.when(slot == b)(functools.partial(fn, b))


def _sc_row_scatter(src, src_idx, dst_idx, n_items, n_out, width):
    info = plsc.get_sparse_core_info()
    NC, NS = info.num_cores, info.num_subcores
    NW = NC * NS
    rows_per_w = n_items // NW
    CH = 32
    nchunk = rows_per_w // CH
    NBUF, H = 3, 2
    mesh = plsc.VectorSubcoreMesh(core_axis_name="c", subcore_axis_name="s")

    @functools.partial(
        pl.kernel, mesh=mesh,
        out_type=jax.ShapeDtypeStruct((n_out, width), jnp.int32),
        scratch_types=[
            pltpu.VMEM((nchunk, 1, CH), jnp.int32),
            pltpu.VMEM((nchunk, 1, CH), jnp.int32),
            pltpu.VMEM((NBUF, CH, width), jnp.int32),
        ] + [pltpu.SemaphoreType.DMA] * (2 * NBUF),
    )
    def scatter_k(src_hbm, gidx_hbm, sidx_hbm, out_hbm, gidx_v, sidx_v,
                  bufs, *sems):
        gsem = sems[:NBUF]
        ssem = sems[NBUF:]
        wid = lax.axis_index("s") * NC + lax.axis_index("c")
        base0 = wid * rows_per_w
        pltpu.sync_copy(gidx_hbm.at[pl.ds(wid * nchunk, nchunk)], gidx_v)
        pltpu.sync_copy(sidx_hbm.at[pl.ds(wid * nchunk, nchunk)], sidx_v)

        def body(v, _):
            @pl.when(v < nchunk)
            def _():
                def start(b):
                    @pl.when(v >= NBUF)
                    def _():
                        pltpu.make_async_copy(
                            bufs.at[b], out_hbm.at[pl.ds(base0, CH)],
                            ssem[b]).wait()
                    pltpu.async_copy(
                        src_hbm.at[gidx_v.at[v, 0]], bufs.at[b], gsem[b])
                _on_slot(lax.rem(v, NBUF), NBUF, start)

            @pl.when(v >= H)
            def _():
                cc = v - H

                def finish(b):
                    pltpu.make_async_copy(
                        src_hbm.at[gidx_v.at[cc, 0]], bufs.at[b],
                        gsem[b]).wait()
                    pltpu.async_copy(
                        bufs.at[b], out_hbm.at[sidx_v.at[cc, 0]], ssem[b])
                _on_slot(lax.rem(cc, NBUF), NBUF, finish)
            return 0

        lax.fori_loop(0, nchunk + H, body, 0)
        for c in range(nchunk - NBUF, nchunk):
            pltpu.make_async_copy(
                bufs.at[c % NBUF], out_hbm.at[pl.ds(base0, CH)],
                ssem[c % NBUF]).wait()

    return scatter_k(src.reshape(-1, width),
                     src_idx.reshape(NW * nchunk, 1, CH),
                     dst_idx.reshape(NW * nchunk, 1, CH))


def _tc_matmul_body(te_ref, xg_ref, w_ref, b_ref, o_ref):
    halfk = xg_ref.shape[1]
    lo, hi = _unpack_halves(xg_ref[...])
    w = w_ref[0]
    mm = lax.dot_general(lo, w[:, :halfk],
                         (((1,), (1,)), ((), ())),
                         preferred_element_type=jnp.float32,
                         precision=lax.Precision.DEFAULT)
    mm = mm + lax.dot_general(hi, w[:, halfk:],
                              (((1,), (1,)), ((), ())),
                              preferred_element_type=jnp.float32,
                              precision=lax.Precision.DEFAULT)
    y = mm + b_ref[0]
    halfo = o_ref.shape[1]
    o_ref[...] = _pack_halves(y[:, :halfo], y[:, halfo:])


def _tc_ragged_matmul(xg_bits, tile_expert, Wb, b, J, NT, D, O, E):
    grid_spec = pltpu.PrefetchScalarGridSpec(
        num_scalar_prefetch=1,
        grid=(NT,),
        in_specs=[
            pl.BlockSpec((TMR, D // 2), lambda i, te: (i, 0)),
            pl.BlockSpec((1, O, D), lambda i, te: (te[i], 0, 0)),
            pl.BlockSpec((1, 1, O), lambda i, te: (te[i], 0, 0)),
        ],
        out_specs=pl.BlockSpec((TMR, O // 2), lambda i, te: (i, 0)),
    )
    return pl.pallas_call(
        _tc_matmul_body,
        grid_spec=grid_spec,
        out_shape=jax.ShapeDtypeStruct((J, O // 2), jnp.int32),
    )(tile_expert, xg_bits, Wb, b.reshape(E, 1, O))


def _tc_combine_body(ew_ref, a_ref, b_ref, o_ref):
    alo, ahi = _unpack_halves(a_ref[...])
    blo, bhi = _unpack_halves(b_ref[...])
    w0 = ew_ref[:, 0:1]
    w1 = ew_ref[:, 1:2]
    half = a_ref.shape[1]
    o_ref[:, :half] = w0 * alo + w1 * blo
    o_ref[:, half:] = w0 * ahi + w1 * bhi


def _tc_combine(ygp, ew, T, O, K):
    TMC = 512
    nblk = T // TMC
    return pl.pallas_call(
        _tc_combine_body,
        grid=(nblk,),
        in_specs=[
            pl.BlockSpec((TMC, K), lambda i: (i, 0)),
            pl.BlockSpec((TMC, O // 2), lambda i: (i, 0)),
            pl.BlockSpec((TMC, O // 2), lambda i, n=nblk: (i + n, 0)),
        ],
        out_specs=pl.BlockSpec((TMC, O), lambda i: (i, 0)),
        out_shape=jax.ShapeDtypeStruct((T, O), jnp.float32),
    )(ew, ygp, ygp)


def kernel(x, expert_weights, top_k_indices, W, b):
    B, S, D = x.shape
    E, O, _ = W.shape
    K = expert_weights.shape[-1]
    T = B * S

    pos, tile_expert, pos_cat, J, NT = _routing_metadata(
        expert_weights, top_k_indices, T, K, E)

    x_bits = _tc_pack(x.reshape(T, D), T, D)
    gidx = jnp.arange(T * K, dtype=jnp.int32) // K
    xg_bits = _sc_row_scatter(x_bits, gidx, pos, T * K, J, D // 2)
    yg_bits = _tc_ragged_matmul(xg_bits, tile_expert, W, b, J, NT, D, O, E)
    ygp = _sc_row_gather(yg_bits, pos_cat, T * K, O // 2)
    out = _tc_combine(ygp, expert_weights.reshape(T, K), T, O, K)
    return out.reshape(B, S, O)
